# baseline (speedup 1.0000x reference)
import functools

import jax
import jax.numpy as jnp
import numpy as np
from jax import lax
from jax.experimental import pallas as pl
from jax.experimental.pallas import tpu as pltpu
from jax.experimental.pallas import tpu_sc as plsc

B, L = 4096, 200
V, D, H = 100000, 128, 64
DW = D // 2
C0, C1 = 104, 96
NC, NS = 2, 16
NW = NC * NS
RPW = B // NW
NBUF = 8
GROUPS = RPW // (NBUF // 2)

_PERM = np.concatenate(
    [np.concatenate([np.arange(16 * c, 16 * c + 16),
                     np.arange(64 + 16 * c, 64 + 16 * c + 16)])
     for c in range(DW // 16)])


def _sc_bag(ids3, tok_packed):
    mesh = plsc.VectorSubcoreMesh(core_axis_name="c", subcore_axis_name="s")

    @functools.partial(
        pl.kernel,
        out_type=jax.ShapeDtypeStruct((B, D), jnp.float32),
        mesh=mesh,
        compiler_params=pltpu.CompilerParams(use_tc_tiling_on_sc=False),
        scratch_types=[
            pltpu.VMEM((RPW, C0), jnp.int32),
            pltpu.VMEM((RPW, C1), jnp.int32),
            pltpu.VMEM((C0, DW), jnp.int32),
            pltpu.VMEM((C1, DW), jnp.int32),
            pltpu.VMEM((C0, DW), jnp.int32),
            pltpu.VMEM((C1, DW), jnp.int32),
            pltpu.VMEM((C0, DW), jnp.int32),
            pltpu.VMEM((C1, DW), jnp.int32),
            pltpu.VMEM((C0, DW), jnp.int32),
            pltpu.VMEM((C1, DW), jnp.int32),
            pltpu.VMEM((RPW, D), jnp.float32),
            pltpu.SemaphoreType.DMA,
            pltpu.SemaphoreType.DMA,
            pltpu.SemaphoreType.DMA,
            pltpu.SemaphoreType.DMA,
            pltpu.SemaphoreType.DMA,
            pltpu.SemaphoreType.DMA,
            pltpu.SemaphoreType.DMA,
            pltpu.SemaphoreType.DMA,
        ],
    )
    def bag(idsa_hbm, idsb_hbm, tok_hbm, out_hbm, ids_va, ids_vb,
            b0, b1, b2, b3, b4, b5, b6, b7,
            out_v, s0, s1, s2, s3, s4, s5, s6, s7):
        wid = lax.axis_index("s") * NC + lax.axis_index("c")
        base = wid * RPW
        pltpu.sync_copy(idsa_hbm.at[pl.ds(base, RPW)], ids_va)
        pltpu.sync_copy(idsb_hbm.at[pl.ds(base, RPW)], ids_vb)
        bufs = (b0, b1, b2, b3, b4, b5, b6, b7)
        sems = (s0, s1, s2, s3, s4, s5, s6, s7)

        def idx_ref(row, half):
            if half:
                return ids_vb.at[row]
            return ids_va.at[row]

        for k in range(NBUF):
            pltpu.async_copy(tok_hbm.at[idx_ref(k // 2, k % 2)],
                             bufs[k], sems[k])

        def accumulate(ref, n, accs):
            def body(i, a):
                base = 8 * i
                out = list(a)
                for u in range(8):
                    for c in range(DW // 16):
                        v = ref[base + u, pl.ds(c * 16, 16)]
                        lo = lax.bitcast_convert_type(v << 16, jnp.float32)
                        hi = lax.bitcast_convert_type(v, jnp.float32)
                        out[2 * c] = out[2 * c] + lo
                        out[2 * c + 1] = out[2 * c + 1] + hi
                return tuple(out)
            return lax.fori_loop(0, n // 8, body, accs)

        def group(g, carry):
            for r in range(NBUF // 2):
                accs = tuple(jnp.zeros((16,), jnp.float32) for _ in range(D // 16))
                row = (NBUF // 2) * g + r
                for half in range(2):
                    k = 2 * r + half
                    n = C1 if half else C0
                    pltpu.make_async_copy(
                        tok_hbm.at[idx_ref(row, half)], bufs[k], sems[k]).wait()
                    accs = accumulate(bufs[k], n, accs)

                    @pl.when(row + NBUF // 2 < RPW)
                    def _():
                        pltpu.async_copy(
                            tok_hbm.at[idx_ref(row + NBUF // 2, half)],
                            bufs[k], sems[k])
                for c8 in range(D // 16):
                    out_v[row, pl.ds(c8 * 16, 16)] = accs[c8]
            return carry

        lax.fori_loop(0, GROUPS, group, 0)
        pltpu.sync_copy(out_v, out_hbm.at[pl.ds(wid * RPW, RPW)])

    return bag(ids3[0], ids3[1], tok_packed)


def _tc_repack(tok_emb):
    bm = 10000
    nblk = (V // 2) // bm

    def body(ta_ref, tb_ref, o_ref):
        a = lax.bitcast_convert_type(ta_ref[...], jnp.int32)
        b = lax.bitcast_convert_type(tb_ref[...], jnp.int32)

        def pack(t):
            return (lax.shift_right_logical(t[:, :DW], 16)
                    | (t[:, DW:] & np.int32(-65536)))

        o_ref[...] = jnp.concatenate([pack(a), pack(b)], axis=1)

    return pl.pallas_call(
        body,
        grid=(nblk,),
        in_specs=[
            pl.BlockSpec((bm, D), lambda i: (i, 0)),
            pl.BlockSpec((bm, D), lambda i: (i + nblk, 0)),
        ],
        out_specs=pl.BlockSpec((bm, D), lambda i: (i, 0)),
        out_shape=jax.ShapeDtypeStruct((V // 2, D), jnp.int32),
    )(tok_emb, tok_emb).reshape(V, DW)


def _tc_pre(x, pos_emb, tok0):
    grid = 16
    blk = B // grid

    def body(x_ref, pos_ref, tok0_ref, rest_ref, inv_ref):
        xb = x_ref[...]
        valid = (xb != 0).astype(jnp.float32)
        cnt = jnp.sum(valid, axis=1, keepdims=True)
        denom = jnp.maximum(cnt, 1.0)
        nzero = jnp.float32(L) - cnt
        pos_term = jnp.dot(valid, pos_ref[...],
                           preferred_element_type=jnp.float32)
        rest_ref[...] = pos_term - nzero * tok0_ref[...]
        inv_ref[...] = 1.0 / denom

    return pl.pallas_call(
        body,
        grid=(grid,),
        in_specs=[
            pl.BlockSpec((blk, L), lambda i: (i, 0)),
            pl.BlockSpec((L, D), lambda i: (0, 0)),
            pl.BlockSpec((1, D), lambda i: (0, 0)),
        ],
        out_specs=[
            pl.BlockSpec((blk, D), lambda i: (i, 0)),
            pl.BlockSpec((blk, 1), lambda i: (i, 0)),
        ],
        out_shape=[
            jax.ShapeDtypeStruct((B, D), jnp.float32),
            jax.ShapeDtypeStruct((B, 1), jnp.float32),
        ],
    )(x, pos_emb, tok0)


def _tc_head(sums, rest, inv, ln_g, ln_b, W1, b1, W2, b2):
    def body(sums_ref, rest_ref, inv_ref, lng_ref, lnb_ref,
             w1_ref, b1_ref, w2_ref, b2_ref, out_ref):
        pooled = (sums_ref[...] + rest_ref[...]) * inv_ref[...]
        mu = jnp.mean(pooled, axis=1, keepdims=True)
        var = jnp.mean((pooled - mu) ** 2, axis=1, keepdims=True)
        hn = (pooled - mu) * lax.rsqrt(var + 1e-5) * lng_ref[...] + lnb_ref[...]
        z = jnp.dot(hn, w1_ref[...], preferred_element_type=jnp.float32)
        z = z + b1_ref[...]
        z = 0.5 * z * (1.0 + lax.erf(z * 0.7071067811865476))
        s = jnp.dot(z, w2_ref[...], preferred_element_type=jnp.float32)
        s = s + b2_ref[...]
        out_ref[...] = s - jnp.mean(s)

    return pl.pallas_call(
        body,
        out_shape=jax.ShapeDtypeStruct((B, 1), jnp.float32),
    )(sums, rest, inv, ln_g, ln_b, W1, b1, W2, b2)


def kernel(x, tok_emb, pos_emb, ln_g, ln_b, W1, b1, W2, b2):
    xm = jnp.where(x < V // 2, 2 * x, 2 * x - (V - 1))
    ids3 = (xm[:, :C0], xm[:, C0:])
    tok_packed = _tc_repack(tok_emb)
    sums = _sc_bag(ids3, tok_packed)
    perm = _PERM
    tok0 = lax.bitcast_convert_type(
        lax.bitcast_convert_type(tok_emb[0:1, perm], jnp.int32)
        & jnp.int32(-65536), jnp.float32)
    rest, inv = _tc_pre(x, pos_emb[:, perm], tok0)
    raw = _tc_head(
        sums, rest, inv,
        ln_g[perm].reshape(1, D), ln_b[perm].reshape(1, D),
        W1[perm, :], b1.reshape(1, H), W2, b2.reshape(1, 1),
    )
    return raw.reshape(B)


# speedup vs baseline: 1.0303x; 1.0303x over previous
import functools

import jax
import jax.numpy as jnp
import numpy as np
from jax import lax
from jax.experimental import pallas as pl
from jax.experimental.pallas import tpu as pltpu
from jax.experimental.pallas import tpu_sc as plsc

B, L = 4096, 200
V, D, H = 100000, 128, 64
DW = D // 2
C0, C1 = 104, 96
NC, NS = 2, 16
NW = NC * NS
RPW = B // NW
NBUF = 8
GROUPS = RPW // (NBUF // 2)

_PERM = np.concatenate(
    [np.concatenate([np.arange(16 * c, 16 * c + 16),
                     np.arange(64 + 16 * c, 64 + 16 * c + 16)])
     for c in range(DW // 16)])


def _sc_bag(ids3, tok_packed):
    mesh = plsc.VectorSubcoreMesh(core_axis_name="c", subcore_axis_name="s")

    @functools.partial(
        pl.kernel,
        out_type=jax.ShapeDtypeStruct((B, D), jnp.float32),
        mesh=mesh,
        compiler_params=pltpu.CompilerParams(use_tc_tiling_on_sc=False),
        scratch_types=[
            pltpu.VMEM((RPW, C0), jnp.int32),
            pltpu.VMEM((RPW, C1), jnp.int32),
            pltpu.VMEM((C0, DW), jnp.int32),
            pltpu.VMEM((C1, DW), jnp.int32),
            pltpu.VMEM((C0, DW), jnp.int32),
            pltpu.VMEM((C1, DW), jnp.int32),
            pltpu.VMEM((C0, DW), jnp.int32),
            pltpu.VMEM((C1, DW), jnp.int32),
            pltpu.VMEM((C0, DW), jnp.int32),
            pltpu.VMEM((C1, DW), jnp.int32),
            pltpu.VMEM((RPW, D), jnp.float32),
            pltpu.SemaphoreType.DMA,
            pltpu.SemaphoreType.DMA,
            pltpu.SemaphoreType.DMA,
            pltpu.SemaphoreType.DMA,
            pltpu.SemaphoreType.DMA,
            pltpu.SemaphoreType.DMA,
            pltpu.SemaphoreType.DMA,
            pltpu.SemaphoreType.DMA,
        ],
    )
    def bag(idsa_hbm, idsb_hbm, tok_hbm, out_hbm, ids_va, ids_vb,
            b0, b1, b2, b3, b4, b5, b6, b7,
            out_v, s0, s1, s2, s3, s4, s5, s6, s7):
        wid = lax.axis_index("s") * NC + lax.axis_index("c")
        base = wid * RPW
        pltpu.sync_copy(idsa_hbm.at[pl.ds(base, RPW)], ids_va)
        pltpu.sync_copy(idsb_hbm.at[pl.ds(base, RPW)], ids_vb)
        bufs = (b0, b1, b2, b3, b4, b5, b6, b7)
        sems = (s0, s1, s2, s3, s4, s5, s6, s7)

        def idx_ref(row, half):
            if half:
                return ids_vb.at[row]
            return ids_va.at[row]

        for k in range(NBUF):
            pltpu.async_copy(tok_hbm.at[idx_ref(k // 2, k % 2)],
                             bufs[k], sems[k])

        def accumulate(ref, n, accs):
            def body(i, a):
                base = 8 * i
                out = list(a)
                for u in range(8):
                    for c in range(DW // 16):
                        v = ref[base + u, pl.ds(c * 16, 16)]
                        lo = lax.bitcast_convert_type(v << 16, jnp.float32)
                        hi = lax.bitcast_convert_type(v, jnp.float32)
                        out[2 * c] = out[2 * c] + lo
                        out[2 * c + 1] = out[2 * c + 1] + hi
                return tuple(out)
            return lax.fori_loop(0, n // 8, body, accs)

        def group(g, carry):
            for r in range(NBUF // 2):
                accs = tuple(jnp.zeros((16,), jnp.float32) for _ in range(D // 16))
                row = (NBUF // 2) * g + r
                for half in range(2):
                    k = 2 * r + half
                    n = C1 if half else C0
                    pltpu.make_async_copy(
                        tok_hbm.at[idx_ref(row, half)], bufs[k], sems[k]).wait()
                    accs = accumulate(bufs[k], n, accs)

                    @pl.when(row + NBUF // 2 < RPW)
                    def _():
                        pltpu.async_copy(
                            tok_hbm.at[idx_ref(row + NBUF // 2, half)],
                            bufs[k], sems[k])
                for c8 in range(D // 16):
                    out_v[row, pl.ds(c8 * 16, 16)] = accs[c8]
            return carry

        lax.fori_loop(0, GROUPS, group, 0)
        pltpu.sync_copy(out_v, out_hbm.at[pl.ds(wid * RPW, RPW)])

    return bag(ids3[0], ids3[1], tok_packed)


def _tc_repack(tok_emb):
    bm = 10000
    nblk = (V // 2) // bm

    def body(ta_ref, tb_ref, o_ref):
        a = lax.bitcast_convert_type(ta_ref[...], jnp.int32)
        b = lax.bitcast_convert_type(tb_ref[...], jnp.int32)

        def pack(t):
            return (lax.shift_right_logical(t[:, :DW], 16)
                    | (t[:, DW:] & np.int32(-65536)))

        o_ref[...] = jnp.concatenate([pack(a), pack(b)], axis=1)

    return pl.pallas_call(
        body,
        grid=(nblk,),
        in_specs=[
            pl.BlockSpec((bm, D), lambda i: (i, 0)),
            pl.BlockSpec((bm, D), lambda i: (i + nblk, 0)),
        ],
        out_specs=pl.BlockSpec((bm, D), lambda i: (i, 0)),
        out_shape=jax.ShapeDtypeStruct((V // 2, D), jnp.int32),
    )(tok_emb, tok_emb).reshape(V, DW)


def _tc_pre(x, pos_emb, tok0):
    grid = 16
    blk = B // grid

    def body(x_ref, pos_ref, tok0_ref, rest_ref, inv_ref):
        xb = x_ref[...]
        valid = (xb != 0).astype(jnp.float32)
        cnt = jnp.sum(valid, axis=1, keepdims=True)
        denom = jnp.maximum(cnt, 1.0)
        nzero = jnp.float32(L) - cnt
        pos_term = jnp.dot(valid, pos_ref[...],
                           preferred_element_type=jnp.float32)
        rest_ref[...] = pos_term - nzero * tok0_ref[...]
        inv_ref[...] = 1.0 / denom

    return pl.pallas_call(
        body,
        grid=(grid,),
        in_specs=[
            pl.BlockSpec((blk, L), lambda i: (i, 0)),
            pl.BlockSpec((L, D), lambda i: (0, 0)),
            pl.BlockSpec((1, D), lambda i: (0, 0)),
        ],
        out_specs=[
            pl.BlockSpec((blk, D), lambda i: (i, 0)),
            pl.BlockSpec((blk, 1), lambda i: (i, 0)),
        ],
        out_shape=[
            jax.ShapeDtypeStruct((B, D), jnp.float32),
            jax.ShapeDtypeStruct((B, 1), jnp.float32),
        ],
    )(x, pos_emb, tok0)


def _tc_head(sums, rest, inv, ln_g, ln_b, W1, b1, W2, b2):
    def body(sums_ref, rest_ref, inv_ref, lng_ref, lnb_ref,
             w1_ref, b1_ref, w2_ref, b2_ref, out_ref):
        pooled = (sums_ref[...] + rest_ref[...]) * inv_ref[...]
        mu = jnp.mean(pooled, axis=1, keepdims=True)
        var = jnp.mean((pooled - mu) ** 2, axis=1, keepdims=True)
        hn = (pooled - mu) * lax.rsqrt(var + 1e-5) * lng_ref[...] + lnb_ref[...]
        z = jnp.dot(hn, w1_ref[...], preferred_element_type=jnp.float32)
        z = z + b1_ref[...]
        z = 0.5 * z * (1.0 + lax.erf(z * 0.7071067811865476))
        s = jnp.dot(z, w2_ref[...], preferred_element_type=jnp.float32)
        s = s + b2_ref[...]
        out_ref[...] = s - jnp.mean(s)

    return pl.pallas_call(
        body,
        out_shape=jax.ShapeDtypeStruct((B, 1), jnp.float32),
    )(sums, rest, inv, ln_g, ln_b, W1, b1, W2, b2)


def kernel(x, tok_emb, pos_emb, ln_g, ln_b, W1, b1, W2, b2):
    xm = jnp.where(x < V // 2, 2 * x, 2 * x - (V - 1))
    ids3 = (xm[:, :C0], xm[:, C0:])
    tok_packed = _tc_repack(tok_emb)
    sums = _sc_bag(ids3, tok_packed)
    perm = _PERM
    r0 = lax.bitcast_convert_type(tok_emb[0], jnp.int32)
    w0 = lax.shift_right_logical(r0[:DW], 16) | (r0[DW:] & jnp.int32(-65536))
    lo0 = lax.bitcast_convert_type(w0 << 16, jnp.float32).reshape(DW // 16, 16)
    hi0 = lax.bitcast_convert_type(w0, jnp.float32).reshape(DW // 16, 16)
    tok0 = jnp.stack([lo0, hi0], axis=1).reshape(1, D)
    rest, inv = _tc_pre(x, pos_emb[:, perm], tok0)
    raw = _tc_head(
        sums, rest, inv,
        ln_g[perm].reshape(1, D), ln_b[perm].reshape(1, D),
        W1[perm, :], b1.reshape(1, H), W2, b2.reshape(1, 1),
    )
    return raw.reshape(B)
